# Initial kernel scaffold; baseline (speedup 1.0000x reference)
import jax
import jax.numpy as jnp
from jax.experimental import pallas as pl


def kernel(atoms, pairs, pair_mask, active, atom_projection, bond_projection, frequency_basis):
    raise NotImplementedError("write your pallas kernel here")


# speedup vs baseline: 1.9459x; 1.9459x over previous
import functools

import jax
import jax.numpy as jnp
from jax import lax
from jax.experimental import pallas as pl
from jax.experimental.pallas import tpu as pltpu
from jax.experimental.pallas import tpu_sc as plsc

DIM_VSA = 2048
DIM_NODE = 27
DIM_NODE_PAD = 32
DIM_EDGE = 12
MAX_EDGES = 128
B = 64
N = 128
TWO_PI = 2.0 * jnp.pi
GRAPHS_PER_WORKER = 2


def _remap_phase(x):
    return x - TWO_PI * jnp.round(x / TWO_PI)


def _sc_body(mask_hbm, atoms_hbm, pairs_hbm,
             src_out, dst_out, edge_out, valid_out,
             mask_v, idx_v, sidx_v, didx_v, pidx_v, pidx2_v, off_v, val_v,
             srow_v, drow_v, prowa_v, prowb_v, erow_v, sem):
    wid = lax.axis_index("s") * 2 + lax.axis_index("c")

    for k in range(GRAPHS_PER_WORKER):
        b = wid * GRAPHS_PER_WORKER + k
        pltpu.sync_copy(mask_hbm.at[b], mask_v)

        fill = jnp.full((16,), N * N - 1, jnp.int32)
        for t in range(10):
            idx_v[pl.ds(t * 16, 16)] = fill

        def row_body(i, cnt):
            c0 = (i + 1) // 16

            def chunk_body(c, cnt):
                m = mask_v[i, pl.ds(c * 16, 16)]
                j16 = lax.iota(jnp.int32, 16) + c * 16
                iv = jnp.broadcast_to(i, (16,))
                cntv = jnp.broadcast_to(cnt, (16,))
                keep = (m != 0.0) & (j16 > iv) & (cntv < MAX_EDGES)
                flat = iv * N + j16
                pcs = plsc.cumsum(keep.astype(jnp.int32))
                pos = jnp.where(keep, cntv + pcs - 1, 159)
                plsc.store_scatter(idx_v, [pos], flat)
                return cnt + jnp.max(pcs)

            return lax.fori_loop(c0, 8, chunk_body, cnt)

        cnt = lax.fori_loop(0, N, row_body, jnp.int32(0))
        cnt = jnp.minimum(cnt, MAX_EDGES)

        n_prow = B * N * N * DIM_EDGE // 16
        for t in range(8):
            fidx = idx_v[pl.ds(t * 16, 16)]
            s = lax.shift_right_logical(fidx, 7)
            d = lax.bitwise_and(fidx, N - 1)
            sidx_v[pl.ds(t * 16, 16)] = b * N + s
            didx_v[pl.ds(t * 16, 16)] = b * N + d
            w = (b * (N * N) + fidx) * DIM_EDGE
            row0 = lax.shift_right_logical(w, 4)
            pidx_v[pl.ds(t * 16, 16)] = row0
            pidx2_v[pl.ds(t * 16, 16)] = jnp.minimum(row0 + 1, n_prow - 1)
            off_v[pl.ds(t * 16, 16)] = lax.bitwise_and(w, 15)
            lane = lax.iota(jnp.int32, 16) + t * 16
            cntv = jnp.broadcast_to(cnt, (16,))
            val_v[pl.ds(t * 16, 16)] = jnp.where(lane < cntv, 1.0, 0.0)

        cp1 = pltpu.async_copy(atoms_hbm.at[sidx_v], srow_v, sem)
        cp2 = pltpu.async_copy(atoms_hbm.at[didx_v], drow_v, sem)
        cp3 = pltpu.async_copy(pairs_hbm.at[pidx_v], prowa_v, sem)
        cp4 = pltpu.async_copy(pairs_hbm.at[pidx2_v], prowb_v, sem)
        cp1.wait()
        cp2.wait()
        cp3.wait()
        cp4.wait()

        for t in range(8):
            e16 = lax.iota(jnp.int32, 16) + t * 16
            off = off_v[pl.ds(t * 16, 16)]
            for c in range(DIM_EDGE):
                pos = off + c
                ja = jnp.minimum(pos, 15)
                jb = jnp.maximum(pos - 16, 0)
                va = plsc.load_gather(prowa_v, [e16, ja])
                vb = plsc.load_gather(prowb_v, [e16, jb])
                v = jnp.where(pos > 15, vb, va)
                cc = jnp.broadcast_to(jnp.int32(c), (16,))
                plsc.store_scatter(erow_v, [e16, cc], v)

        pltpu.sync_copy(srow_v, src_out.at[b])
        pltpu.sync_copy(drow_v, dst_out.at[b])
        pltpu.sync_copy(erow_v, edge_out.at[b])
        pltpu.sync_copy(val_v, valid_out.at[b])


def _sc_extract(pair_mask, atoms_flat, pairs_flat):
    mesh = plsc.VectorSubcoreMesh(core_axis_name="c", subcore_axis_name="s")
    f32 = jnp.float32
    run = pl.kernel(
        _sc_body,
        out_type=(
            jax.ShapeDtypeStruct((B, MAX_EDGES, DIM_NODE_PAD), f32),
            jax.ShapeDtypeStruct((B, MAX_EDGES, DIM_NODE_PAD), f32),
            jax.ShapeDtypeStruct((B, MAX_EDGES, DIM_EDGE), f32),
            jax.ShapeDtypeStruct((B, MAX_EDGES), f32),
        ),
        mesh=mesh,
        scratch_types=(
            pltpu.VMEM((N, N), f32),
            pltpu.VMEM((160,), jnp.int32),
            pltpu.VMEM((MAX_EDGES,), jnp.int32),
            pltpu.VMEM((MAX_EDGES,), jnp.int32),
            pltpu.VMEM((MAX_EDGES,), jnp.int32),
            pltpu.VMEM((MAX_EDGES,), jnp.int32),
            pltpu.VMEM((MAX_EDGES,), jnp.int32),
            pltpu.VMEM((MAX_EDGES,), f32),
            pltpu.VMEM((MAX_EDGES, DIM_NODE_PAD), f32),
            pltpu.VMEM((MAX_EDGES, DIM_NODE_PAD), f32),
            pltpu.VMEM((MAX_EDGES, 16), f32),
            pltpu.VMEM((MAX_EDGES, 16), f32),
            pltpu.VMEM((MAX_EDGES, DIM_EDGE), f32),
            pltpu.SemaphoreType.DMA,
        ),
        compiler_params=pltpu.CompilerParams(
            needs_layout_passes=False, use_tc_tiling_on_sc=False),
    )
    return run(pair_mask, atoms_flat, pairs_flat)


def _tc_body(s_ref, d_ref, e_ref, v_ref, pa_ref, pb_ref, f_ref, out_ref):
    sp = _remap_phase(jnp.dot(s_ref[0], pa_ref[...],
                              preferred_element_type=jnp.float32))
    dp = _remap_phase(jnp.dot(d_ref[0], pa_ref[...],
                              preferred_element_type=jnp.float32))
    ep = _remap_phase(jnp.dot(e_ref[0], pb_ref[...],
                              preferred_element_type=jnp.float32))
    g = _remap_phase(sp + dp + ep)
    pos = lax.broadcasted_iota(jnp.int32, (MAX_EDGES, DIM_VSA), 0).astype(
        jnp.float32)
    spec = _remap_phase(pos * f_ref[...])
    g = _remap_phase(g + spec)
    out_ref[0] = g * v_ref[0]


def _tc_compute(src_rows, dst_rows, edge_rows, valid3, pa_pad, pb, fb):
    grid = (B,)
    return pl.pallas_call(
        _tc_body,
        grid=grid,
        in_specs=[
            pl.BlockSpec((1, MAX_EDGES, DIM_NODE_PAD), lambda g: (g, 0, 0)),
            pl.BlockSpec((1, MAX_EDGES, DIM_NODE_PAD), lambda g: (g, 0, 0)),
            pl.BlockSpec((1, MAX_EDGES, DIM_EDGE), lambda g: (g, 0, 0)),
            pl.BlockSpec((1, MAX_EDGES, 1), lambda g: (g, 0, 0)),
            pl.BlockSpec((DIM_NODE_PAD, DIM_VSA), lambda g: (0, 0)),
            pl.BlockSpec((DIM_EDGE, DIM_VSA), lambda g: (0, 0)),
            pl.BlockSpec((1, DIM_VSA), lambda g: (0, 0)),
        ],
        out_specs=pl.BlockSpec((1, MAX_EDGES, DIM_VSA), lambda g: (g, 0, 0)),
        out_shape=jax.ShapeDtypeStruct((B, MAX_EDGES, DIM_VSA), jnp.float32),
        compiler_params=pltpu.CompilerParams(
            dimension_semantics=("arbitrary",),
        ),
    )(src_rows, dst_rows, edge_rows, valid3, pa_pad, pb, fb)


def kernel(atoms, pairs, pair_mask, active, atom_projection, bond_projection,
           frequency_basis):
    atoms_pad = jnp.pad(atoms, ((0, 0), (0, 0), (0, DIM_NODE_PAD - DIM_NODE)))
    atoms_flat = atoms_pad.reshape(B * N, DIM_NODE_PAD)
    pairs_flat = pairs.reshape(B * N * N * DIM_EDGE // 16, 16)
    pa_pad = jnp.pad(atom_projection, ((0, DIM_NODE_PAD - DIM_NODE), (0, 0)))

    src_rows, dst_rows, edge_rows, valid = _sc_extract(
        pair_mask, atoms_flat, pairs_flat)
    valid3 = valid.reshape(B, MAX_EDGES, 1)

    return _tc_compute(src_rows, dst_rows, edge_rows, valid3, pa_pad,
                       bond_projection, frequency_basis)
